# baseline (speedup 1.0000x reference)
import jax
import jax.numpy as jnp
from jax import lax
from jax.experimental import pallas as pl
from jax.experimental.pallas import tpu as pltpu
from jax.experimental.pallas import tpu_sc as plsc

DX, DY, DZ = 24, 24, 16
DOUT = DX + DY + DZ
NC, NS = 2, 16
NW = NC * NS
CHUNK = 512
L = 16


def _body(idx_hbm, tx_hbm, ty_hbm, tz_hbm, out_hbm,
          idx_v, tx_v, ty_v, tz_v, out_v):
    n = out_hbm.shape[0] // DOUT
    tok_per_w = n // NW
    n_chunks = tok_per_w // CHUNK
    wid = lax.axis_index("s") * NC + lax.axis_index("c")
    base_w = wid * tok_per_w
    pltpu.sync_copy(tx_hbm, tx_v)
    pltpu.sync_copy(ty_hbm, ty_v)
    pltpu.sync_copy(tz_hbm, tz_v)
    lane = jax.lax.iota(jnp.int32, L)
    lane3 = lane * 3
    lane64 = lane * DOUT

    def group(s, carry):
        tri = lane3 + s * (3 * L)
        ixg = plsc.load_gather(idx_v, [tri]) * DX
        iyg = plsc.load_gather(idx_v, [tri + 1]) * DY
        izg = plsc.load_gather(idx_v, [tri + 2]) * DZ
        obase = lane64 + s * (L * DOUT)
        for c in range(DX):
            plsc.store_scatter(out_v, [obase + c],
                               plsc.load_gather(tx_v, [ixg + c]))
        for c in range(DY):
            plsc.store_scatter(out_v, [obase + (DX + c)],
                               plsc.load_gather(ty_v, [iyg + (8 + c)]))
        for c in range(DZ):
            plsc.store_scatter(out_v, [obase + (DX + DY + c)],
                               plsc.load_gather(tz_v, [izg + c]))
        return carry

    @pl.loop(0, n_chunks)
    def chunk_loop(j):
        base = pl.multiple_of(base_w + j * CHUNK, CHUNK)
        pltpu.sync_copy(idx_hbm.at[pl.ds(base * 3, CHUNK * 3)], idx_v)
        lax.fori_loop(0, CHUNK // L, group, 0)
        pltpu.sync_copy(out_v, out_hbm.at[pl.ds(base * DOUT, CHUNK * DOUT)])


@jax.jit
def kernel(indices, cis_x, cis_y, cis_z):
    n = indices.shape[0]
    idx = indices.reshape(-1)
    tx = jnp.pad(cis_x.reshape(-1), (0, 8))
    ty = jnp.pad(cis_y.reshape(-1), (8, 0))
    tz = cis_z.reshape(-1)
    mesh = plsc.VectorSubcoreMesh(core_axis_name="c", subcore_axis_name="s",
                                  num_cores=NC, num_subcores=NS)
    out = pl.kernel(
        _body,
        out_type=jax.ShapeDtypeStruct((n * DOUT,), jnp.float32),
        mesh=mesh,
        scratch_types=[
            pltpu.VMEM((CHUNK * 3,), jnp.int32),
            pltpu.VMEM((1024 * DX + 8,), jnp.float32),
            pltpu.VMEM((1024 * DY + 8,), jnp.float32),
            pltpu.VMEM((512 * DZ,), jnp.float32),
            pltpu.VMEM((CHUNK * DOUT,), jnp.float32),
        ],
        compiler_params=pltpu.CompilerParams(needs_layout_passes=False),
    )(idx, tx, ty, tz)
    return out.reshape(1, n, DOUT // 2, 2)


# speedup vs baseline: 21.0212x; 21.0212x over previous
import jax
import jax.numpy as jnp
from jax import lax
from jax.experimental import pallas as pl
from jax.experimental.pallas import tpu as pltpu
from jax.experimental.pallas import tpu_sc as plsc

DX, DY, DZ = 24, 24, 16
DOUT = DX + DY + DZ
NF = DOUT // 2
NC, NS = 2, 16
NW = NC * NS
CHUNK = 512
BLK = 128
SEG = (CHUNK // BLK) * 2 * BLK
L = 16


def _body(ix_hbm, iy_hbm, iz_hbm, tx_hbm, ty_hbm, tz_hbm, out_hbm,
          ix_v, iy_v, iz_v, tx_v, ty_v, tz_v, out_v, sem):
    n = out_hbm.shape[0] // DOUT
    tok_per_w = n // NW
    n_chunks = tok_per_w // CHUNK
    wid = lax.axis_index("s") * NC + lax.axis_index("c")
    base_w = wid * tok_per_w
    pltpu.sync_copy(tx_hbm, tx_v)
    pltpu.sync_copy(ty_hbm, ty_v)
    pltpu.sync_copy(tz_hbm, tz_v)

    def group(s, carry):
        ixg = ix_v[pl.ds(s * L, L)]
        iyg = iy_v[pl.ds(s * L, L)]
        izg = iz_v[pl.ds(s * L, L)]
        obase = (s // 8) * (2 * BLK) + (s % 8) * L
        for w in range(DOUT):
            if w < DX:
                v = plsc.load_gather(tx_v, [ixg + w])
            elif w < DX + DY:
                v = plsc.load_gather(ty_v, [iyg + (w - DX)])
            else:
                v = plsc.load_gather(tz_v, [izg + (w - DX - DY)])
            f, r = w // 2, w % 2
            out_v[pl.ds(f * SEG + r * BLK + obase, L)] = v
        return carry

    @pl.loop(0, n_chunks)
    def chunk_loop(j):
        base = pl.multiple_of(base_w + j * CHUNK, CHUNK)
        pltpu.sync_copy(ix_hbm.at[pl.ds(base, CHUNK)], ix_v)
        pltpu.sync_copy(iy_hbm.at[pl.ds(base, CHUNK)], iy_v)
        pltpu.sync_copy(iz_hbm.at[pl.ds(base, CHUNK)], iz_v)
        lax.fori_loop(0, CHUNK // L, group, 0)
        copies = []
        for f in range(NF):
            copies.append(pltpu.async_copy(
                out_v.at[pl.ds(f * SEG, SEG)],
                out_hbm.at[pl.ds(f * (2 * n) + base * 2, SEG)], sem))
        for c in copies:
            c.wait()


@jax.jit
def kernel(indices, cis_x, cis_y, cis_z):
    n = indices.shape[0]
    ix = indices[:, 0] * DX
    iy = indices[:, 1] * DY
    iz = indices[:, 2] * DZ
    tx = cis_x.reshape(-1)
    ty = cis_y.reshape(-1)
    tz = cis_z.reshape(-1)
    mesh = plsc.VectorSubcoreMesh(core_axis_name="c", subcore_axis_name="s",
                                  num_cores=NC, num_subcores=NS)
    out = pl.kernel(
        _body,
        out_type=jax.ShapeDtypeStruct((n * DOUT,), jnp.float32),
        mesh=mesh,
        scratch_types=[
            pltpu.VMEM((CHUNK,), jnp.int32),
            pltpu.VMEM((CHUNK,), jnp.int32),
            pltpu.VMEM((CHUNK,), jnp.int32),
            pltpu.VMEM((1024 * DX,), jnp.float32),
            pltpu.VMEM((1024 * DY,), jnp.float32),
            pltpu.VMEM((512 * DZ,), jnp.float32),
            pltpu.VMEM((CHUNK * DOUT,), jnp.float32),
            pltpu.SemaphoreType.DMA,
        ],
        compiler_params=pltpu.CompilerParams(needs_layout_passes=False),
    )(ix, iy, iz, tx, ty, tz)
    o4 = out.reshape(NF, n // BLK, 2, BLK)
    return o4.transpose(1, 3, 0, 2).reshape(n, NF, 2)[None]


# speedup vs baseline: 22.5376x; 1.0721x over previous
import jax
import jax.numpy as jnp
from jax import lax
from jax.experimental import pallas as pl
from jax.experimental.pallas import tpu as pltpu
from jax.experimental.pallas import tpu_sc as plsc

DX, DY, DZ = 24, 24, 16
DOUT = DX + DY + DZ
NF = DOUT // 2
NC, NS = 2, 16
NW = NC * NS
CHUNK = 1024
BLK = 128
SEG = (CHUNK // BLK) * 2 * BLK
L = 16


def _body(ix_hbm, iy_hbm, iz_hbm, tx_hbm, ty_hbm, tz_hbm, out_hbm,
          ix_v, iy_v, iz_v, tx_v, ty_v, tz_v, out_v, sem):
    n = out_hbm.shape[0] // DOUT
    tok_per_w = n // NW
    n_chunks = tok_per_w // CHUNK
    wid = lax.axis_index("s") * NC + lax.axis_index("c")
    base_w = wid * tok_per_w
    pltpu.sync_copy(tx_hbm, tx_v)
    pltpu.sync_copy(ty_hbm, ty_v)
    pltpu.sync_copy(tz_hbm, tz_v)

    def group(s, carry):
        ixg = ix_v[pl.ds(s * L, L)]
        iyg = iy_v[pl.ds(s * L, L)]
        izg = iz_v[pl.ds(s * L, L)]
        obase = (s // 8) * (2 * BLK) + (s % 8) * L
        for w in range(DOUT):
            if w < DX:
                v = plsc.load_gather(tx_v, [ixg + w])
            elif w < DX + DY:
                v = plsc.load_gather(ty_v, [iyg + (w - DX)])
            else:
                v = plsc.load_gather(tz_v, [izg + (w - DX - DY)])
            f, r = w // 2, w % 2
            out_v[pl.ds(f * SEG + r * BLK + obase, L)] = v
        return carry

    @pl.loop(0, n_chunks)
    def chunk_loop(j):
        base = pl.multiple_of(base_w + j * CHUNK, CHUNK)
        ci = pltpu.async_copy(ix_hbm.at[pl.ds(base, CHUNK)], ix_v, sem)
        cy = pltpu.async_copy(iy_hbm.at[pl.ds(base, CHUNK)], iy_v, sem)
        cz = pltpu.async_copy(iz_hbm.at[pl.ds(base, CHUNK)], iz_v, sem)
        ci.wait()
        cy.wait()
        cz.wait()
        lax.fori_loop(0, CHUNK // L, group, 0)
        copies = []
        for f in range(NF):
            copies.append(pltpu.async_copy(
                out_v.at[pl.ds(f * SEG, SEG)],
                out_hbm.at[pl.ds(f * (2 * n) + base * 2, SEG)], sem))
        for c in copies:
            c.wait()


@jax.jit
def kernel(indices, cis_x, cis_y, cis_z):
    n = indices.shape[0]
    ix = indices[:, 0] * DX
    iy = indices[:, 1] * DY
    iz = indices[:, 2] * DZ
    tx = cis_x.reshape(-1)
    ty = cis_y.reshape(-1)
    tz = cis_z.reshape(-1)
    mesh = plsc.VectorSubcoreMesh(core_axis_name="c", subcore_axis_name="s",
                                  num_cores=NC, num_subcores=NS)
    out = pl.kernel(
        _body,
        out_type=jax.ShapeDtypeStruct((n * DOUT,), jnp.float32),
        mesh=mesh,
        scratch_types=[
            pltpu.VMEM((CHUNK,), jnp.int32),
            pltpu.VMEM((CHUNK,), jnp.int32),
            pltpu.VMEM((CHUNK,), jnp.int32),
            pltpu.VMEM((1024 * DX,), jnp.float32),
            pltpu.VMEM((1024 * DY,), jnp.float32),
            pltpu.VMEM((512 * DZ,), jnp.float32),
            pltpu.VMEM((CHUNK * DOUT,), jnp.float32),
            pltpu.SemaphoreType.DMA,
        ],
        compiler_params=pltpu.CompilerParams(needs_layout_passes=False),
    )(ix, iy, iz, tx, ty, tz)
    o4 = out.reshape(NF, n // BLK, 2, BLK)
    return o4.transpose(1, 3, 0, 2).reshape(n, NF, 2)[None]
